# baseline (speedup 1.0000x reference)
import functools

import jax
import jax.numpy as jnp
from jax.experimental import pallas as pl
from jax.experimental.pallas import tpu as pltpu

_BATCH = 1024
_D = 128
_V = 100000
_KV = 2048
_NV = (_V + _KV - 1) // _KV


def _lsm_kernel(x_ref, w_ref, b_ref, out_ref, m_ref, s_ref):
    p = pl.program_id(0)
    j = pl.program_id(1)

    x = x_ref[...]
    w = w_ref[...]
    logits = jax.lax.dot_general(
        x, w, (((1,), (1,)), ((), ())), preferred_element_type=jnp.float32
    ) + b_ref[...]

    cols = jax.lax.broadcasted_iota(jnp.int32, (1, _KV), 1) + j * _KV
    masked = jnp.where(cols < _V, logits, -jnp.inf)

    @pl.when(p == 0)
    def _accumulate():
        tile_m = jnp.max(masked, axis=1, keepdims=True)
        tile_s = jnp.sum(jnp.exp(masked - tile_m), axis=1, keepdims=True)

        @pl.when(j == 0)
        def _init():
            m_ref[...] = tile_m
            s_ref[...] = tile_s

        @pl.when(j > 0)
        def _update():
            m_old = m_ref[...]
            m_new = jnp.maximum(m_old, tile_m)
            s_ref[...] = (s_ref[...] * jnp.exp(m_old - m_new)
                          + tile_s * jnp.exp(tile_m - m_new))
            m_ref[...] = m_new

    @pl.when(p == 1)
    def _write():
        lse = m_ref[...] + jnp.log(s_ref[...])
        out_ref[...] = masked - lse


@functools.partial(jax.jit, static_argnames=())
def kernel(inputs, W, b):
    b2 = b.reshape(1, _V)
    out = pl.pallas_call(
        _lsm_kernel,
        grid=(2, _NV),
        in_specs=[
            pl.BlockSpec((_BATCH, _D), lambda p, j: (0, 0)),
            pl.BlockSpec((_KV, _D), lambda p, j: (j, 0)),
            pl.BlockSpec((1, _KV), lambda p, j: (0, j)),
        ],
        out_specs=pl.BlockSpec(
            (_BATCH, _KV),
            lambda p, j: (0, jax.lax.select(p > 0, j, 0)),
        ),
        out_shape=jax.ShapeDtypeStruct((_BATCH, _V), jnp.float32),
        scratch_shapes=[
            pltpu.VMEM((_BATCH, 1), jnp.float32),
            pltpu.VMEM((_BATCH, 1), jnp.float32),
        ],
    )(inputs, W, b2)
    return out
